# baseline (speedup 1.0000x reference)
import functools

import jax
import jax.numpy as jnp
from jax import lax
from jax.experimental import pallas as pl
from jax.experimental.pallas import tpu as pltpu
from jax.experimental.pallas import tpu_sc as plsc

N_INF = 1024
N_VEH = 4096
D = 256
BV = 1024
GRID_A = N_VEH // BV
BIG = 1e6

_NC = 2
_NS = 16
_NW = _NC * _NS
_FPW = N_INF // _NW
_CPW = (N_VEH - N_INF) // _NW


def _stage_a(inf_abs_ref, veh_pts_ref, veh_dims_ref, sc_row_ref, sc_col_ref,
             infq_ref, wt_ref, b_ref,
             bestval_ref, bestidx_ref, ranks_ref, addv_ref):
    g = pl.program_id(0)
    tx = inf_abs_ref[:, 0:1]
    ty = inf_abs_ref[:, 1:2]
    tz = inf_abs_ref[:, 2:3]
    vx = veh_pts_ref[0:1, :] * 102.4 - 51.2
    vy = veh_pts_ref[1:2, :] * 102.4 - 51.2
    vz = veh_pts_ref[2:3, :] * 8.0 - 5.0
    dx = vx - tx
    dy = vy - ty
    dz = vz - tz
    dist = jnp.sqrt(dx * dx + dy * dy + dz * dz + 1e-12)
    dmx = jnp.exp(veh_dims_ref[0:1, :])
    dmy = jnp.exp(veh_dims_ref[1:2, :])
    dmz = jnp.exp(veh_dims_ref[2:3, :])
    ok = ((jnp.abs(dx) / dmx <= 1.0)
          & (jnp.abs(dy) / dmy <= 1.0)
          & (jnp.abs(dz) / dmz <= 1.0))
    svz = sc_row_ref[...]
    cost = jnp.where((svz >= 0.05) & ok, dist, BIG)
    m = jnp.min(cost, axis=1, keepdims=True)
    lanes = lax.broadcasted_iota(jnp.int32, (N_INF, BV), 1)
    idx = jnp.min(jnp.where(cost == m, lanes, N_VEH), axis=1,
                  keepdims=True) + g * BV

    @pl.when(g == 0)
    def _():
        bestval_ref[...] = m
        bestidx_ref[...] = idx

    @pl.when(g > 0)
    def _():
        prev = bestval_ref[...]
        better = m < prev
        bestidx_ref[...] = jnp.where(better, idx, bestidx_ref[...])
        bestval_ref[...] = jnp.where(better, m, prev)

    sj = sc_col_ref[...]
    jj = lax.broadcasted_iota(jnp.int32, (N_VEH, BV), 0)
    ii = lax.broadcasted_iota(jnp.int32, (N_VEH, BV), 1) + g * BV
    cmp = (sj > svz) | ((sj == svz) & (jj < ii))
    ranks_ref[...] = jnp.sum(cmp.astype(jnp.int32), axis=0, keepdims=True)

    @pl.when(g == GRID_A - 1)
    def _():
        proj = jnp.dot(infq_ref[...], wt_ref[...],
                       preferred_element_type=jnp.float32,
                       precision=lax.Precision.HIGHEST) + b_ref[...]
        accept = bestval_ref[...] < 1e5
        addv_ref[...] = jnp.where(accept, proj, 0.0)


def _stage_c(vfeats_hbm, vidx_hbm, ranks_hbm, addv_hbm, out_hbm,
             idxf_v, ranks_v, idxc_v, rowsf_v, rowsc_v, add_v,
             semi, semr, sema, semf, semc):
    wid = lax.axis_index("s") * _NC + lax.axis_index("c")
    fbase = wid * _FPW
    cbase = wid * _CPW
    ci = pltpu.async_copy(vidx_hbm.at[pl.ds(fbase, _FPW)], idxf_v, semi)
    cr = pltpu.async_copy(ranks_hbm.at[:], ranks_v, semr)
    ca = pltpu.async_copy(addv_hbm.at[pl.ds(fbase, _FPW)], add_v, sema)
    ci.wait()
    cf = pltpu.async_copy(vfeats_hbm.at[idxf_v], rowsf_v, semf)
    cr.wait()

    def _perm_step(t, _):
        r = ranks_v[pl.ds(t * 16, 16)] - cbase
        vals = lax.broadcasted_iota(jnp.int32, (16,), 0) + t * 16
        mask = (r >= 0) & (r < _CPW)
        r = jnp.where(mask, r, 0)
        plsc.store_scatter(idxc_v, [r], vals, mask=mask)
        return _

    lax.fori_loop(0, N_VEH // 16, _perm_step, 0)
    cc = pltpu.async_copy(vfeats_hbm.at[idxc_v], rowsc_v, semc)
    cf.wait()
    ca.wait()

    def _add_row(r, _):
        for j in range(D // 16):
            sl = pl.ds(j * 16, 16)
            rowsf_v[r, sl] = rowsf_v[r, sl] + add_v[r, sl]
        return _

    lax.fori_loop(0, _FPW, _add_row, 0)
    pltpu.sync_copy(rowsf_v, out_hbm.at[pl.ds(fbase, _FPW)])
    cc.wait()
    pltpu.sync_copy(rowsc_v, out_hbm.at[pl.ds(N_INF + cbase, _CPW)])


def kernel(inf_ref_pts, inf_query_feats, veh_ref_pts, veh_query_feats,
           veh_scores, veh_pred_dims, veh2inf_rt, W_fusion, b_fusion):
    calib = jnp.linalg.inv(veh2inf_rt.T)
    _pts = jnp.concatenate([inf_ref_pts[:, 0:1] * 102.4 - 51.2,
                            inf_ref_pts[:, 1:2] * 102.4 - 51.2,
                            inf_ref_pts[:, 2:3] * 8.0 - 5.0], axis=1)
    _homo = jnp.concatenate([_pts, jnp.ones((N_INF, 1), jnp.float32)], axis=1)
    inf_abs = (_homo @ calib.T)[:, :3]

    bestval, bestidx, ranks, addv = pl.pallas_call(
        _stage_a,
        grid=(GRID_A,),
        in_specs=[
            pl.BlockSpec((N_INF, 3), lambda g: (0, 0)),
            pl.BlockSpec((3, BV), lambda g: (0, g)),
            pl.BlockSpec((3, BV), lambda g: (0, g)),
            pl.BlockSpec((1, BV), lambda g: (0, g)),
            pl.BlockSpec((N_VEH, 1), lambda g: (0, 0)),
            pl.BlockSpec((N_INF, D), lambda g: (0, 0)),
            pl.BlockSpec((D, D), lambda g: (0, 0)),
            pl.BlockSpec((1, D), lambda g: (0, 0)),
        ],
        out_specs=[
            pl.BlockSpec((N_INF, 1), lambda g: (0, 0)),
            pl.BlockSpec((N_INF, 1), lambda g: (0, 0)),
            pl.BlockSpec((1, BV), lambda g: (0, g)),
            pl.BlockSpec((N_INF, D), lambda g: (0, 0)),
        ],
        out_shape=[
            jax.ShapeDtypeStruct((N_INF, 1), jnp.float32),
            jax.ShapeDtypeStruct((N_INF, 1), jnp.int32),
            jax.ShapeDtypeStruct((1, N_VEH), jnp.int32),
            jax.ShapeDtypeStruct((N_INF, D), jnp.float32),
        ],
    )(inf_abs, veh_ref_pts.T, veh_pred_dims.T, veh_scores.reshape(1, N_VEH),
      veh_scores.reshape(N_VEH, 1), inf_query_feats, W_fusion.T,
      b_fusion.reshape(1, D))

    sc_kernel = functools.partial(
        pl.kernel,
        out_type=jax.ShapeDtypeStruct((N_VEH, D), jnp.float32),
        mesh=plsc.VectorSubcoreMesh(core_axis_name="c", subcore_axis_name="s"),
        compiler_params=pltpu.CompilerParams(needs_layout_passes=False),
        scratch_types=[
            pltpu.VMEM((_FPW,), jnp.int32),
            pltpu.VMEM((N_VEH,), jnp.int32),
            pltpu.VMEM((_CPW,), jnp.int32),
            pltpu.VMEM((_FPW, D), jnp.float32),
            pltpu.VMEM((_CPW, D), jnp.float32),
            pltpu.VMEM((_FPW, D), jnp.float32),
            pltpu.SemaphoreType.DMA,
            pltpu.SemaphoreType.DMA,
            pltpu.SemaphoreType.DMA,
            pltpu.SemaphoreType.DMA,
            pltpu.SemaphoreType.DMA,
        ],
    )(_stage_c)
    out = sc_kernel(veh_query_feats, bestidx.reshape(N_INF),
                    ranks.reshape(N_VEH), addv)
    return out


# speedup vs baseline: 1.0876x; 1.0876x over previous
import functools

import jax
import jax.numpy as jnp
from jax import lax
from jax.experimental import pallas as pl
from jax.experimental.pallas import tpu as pltpu
from jax.experimental.pallas import tpu_sc as plsc

N_INF = 1024
N_VEH = 4096
D = 256
BV = 512
GRID_A = N_VEH // BV
BIG = 1e6

_NC = 2
_NS = 16
_NW = _NC * _NS
_FPW = N_INF // _NW
_CPW = (N_VEH - N_INF) // _NW


def _stage_ranks(sc_col_ref, sc_row_ref, ranks_ref):
    g = pl.program_id(0)
    si = sc_col_ref[...]
    sj = sc_row_ref[...]
    jj = lax.broadcasted_iota(jnp.int32, (BV, N_VEH), 1)
    ii = lax.broadcasted_iota(jnp.int32, (BV, N_VEH), 0) + g * BV
    cmp = (sj > si) | ((sj == si) & (jj < ii))
    ranks_ref[...] = jnp.sum(cmp.astype(jnp.int32), axis=1, keepdims=True)


def _stage_cost(inf_abs_ref, veh_pts_ref, veh_dims_ref, sc_row_ref,
                infq_ref, wt_ref, b_ref,
                bestval_ref, bestidx_ref, addv_ref):
    g = pl.program_id(0)
    tx = inf_abs_ref[:, 0:1]
    ty = inf_abs_ref[:, 1:2]
    tz = inf_abs_ref[:, 2:3]
    vx = veh_pts_ref[0:1, :] * 102.4 - 51.2
    vy = veh_pts_ref[1:2, :] * 102.4 - 51.2
    vz = veh_pts_ref[2:3, :] * 8.0 - 5.0
    dx = vx - tx
    dy = vy - ty
    dz = vz - tz
    dist = jnp.sqrt(dx * dx + dy * dy + dz * dz + 1e-12)
    dmx = jnp.exp(veh_dims_ref[0:1, :])
    dmy = jnp.exp(veh_dims_ref[1:2, :])
    dmz = jnp.exp(veh_dims_ref[2:3, :])
    ok = ((jnp.abs(dx) / dmx <= 1.0)
          & (jnp.abs(dy) / dmy <= 1.0)
          & (jnp.abs(dz) / dmz <= 1.0))
    svz = sc_row_ref[...]
    cost = jnp.where((svz >= 0.05) & ok, dist, BIG)
    m = jnp.min(cost, axis=1, keepdims=True)
    lanes = lax.broadcasted_iota(jnp.int32, (N_INF, BV), 1)
    idx = jnp.min(jnp.where(cost == m, lanes, N_VEH), axis=1,
                  keepdims=True) + g * BV

    @pl.when(g == 0)
    def _():
        bestval_ref[...] = m
        bestidx_ref[...] = idx

    @pl.when(g > 0)
    def _():
        prev = bestval_ref[...]
        better = m < prev
        bestidx_ref[...] = jnp.where(better, idx, bestidx_ref[...])
        bestval_ref[...] = jnp.where(better, m, prev)

    @pl.when(g == GRID_A - 1)
    def _():
        proj = jnp.dot(infq_ref[...], wt_ref[...],
                       preferred_element_type=jnp.float32,
                       precision=lax.Precision.HIGHEST) + b_ref[...]
        accept = bestval_ref[...] < 1e5
        addv_ref[...] = jnp.where(accept, proj, 0.0)


def _sc_comp(vfeats_hbm, ranks_hbm, out_hbm, ranks_v, idxc_v, rowsc_v, semc):
    wid = lax.axis_index("s") * _NC + lax.axis_index("c")
    cbase = wid * _CPW
    pltpu.sync_copy(ranks_hbm.at[:], ranks_v)

    def _perm_step(t, _):
        r = ranks_v[pl.ds(t * 16, 16)] - cbase
        vals = lax.broadcasted_iota(jnp.int32, (16,), 0) + t * 16
        mask = (r >= 0) & (r < _CPW)
        r = jnp.where(mask, r, 0)
        plsc.store_scatter(idxc_v, [r], vals, mask=mask)
        return _

    lax.fori_loop(0, N_VEH // 16, _perm_step, 0)
    pltpu.async_copy(vfeats_hbm.at[idxc_v], rowsc_v, semc).wait()
    pltpu.sync_copy(rowsc_v, out_hbm.at[pl.ds(cbase, _CPW)])


def _sc_fused(vfeats_hbm, vidx_hbm, addv_hbm, out_hbm,
              idxf_v, rowsf_v, add_v, semi, sema, semf):
    wid = lax.axis_index("s") * _NC + lax.axis_index("c")
    fbase = wid * _FPW
    ci = pltpu.async_copy(vidx_hbm.at[pl.ds(fbase, _FPW)], idxf_v, semi)
    ca = pltpu.async_copy(addv_hbm.at[pl.ds(fbase, _FPW)], add_v, sema)
    ci.wait()
    cf = pltpu.async_copy(vfeats_hbm.at[idxf_v], rowsf_v, semf)
    cf.wait()
    ca.wait()

    def _add_row(r, _):
        for j in range(D // 16):
            sl = pl.ds(j * 16, 16)
            rowsf_v[r, sl] = rowsf_v[r, sl] + add_v[r, sl]
        return _

    lax.fori_loop(0, _FPW, _add_row, 0)
    pltpu.sync_copy(rowsf_v, out_hbm.at[pl.ds(fbase, _FPW)])


def kernel(inf_ref_pts, inf_query_feats, veh_ref_pts, veh_query_feats,
           veh_scores, veh_pred_dims, veh2inf_rt, W_fusion, b_fusion):
    calib = jnp.linalg.inv(veh2inf_rt.T)
    _pts = jnp.concatenate([inf_ref_pts[:, 0:1] * 102.4 - 51.2,
                            inf_ref_pts[:, 1:2] * 102.4 - 51.2,
                            inf_ref_pts[:, 2:3] * 8.0 - 5.0], axis=1)
    _homo = jnp.concatenate([_pts, jnp.ones((N_INF, 1), jnp.float32)], axis=1)
    inf_abs = (_homo @ calib.T)[:, :3]

    ranks = pl.pallas_call(
        _stage_ranks,
        grid=(GRID_A,),
        in_specs=[
            pl.BlockSpec((BV, 1), lambda g: (g, 0)),
            pl.BlockSpec((1, N_VEH), lambda g: (0, 0)),
        ],
        out_specs=pl.BlockSpec((BV, 1), lambda g: (g, 0)),
        out_shape=jax.ShapeDtypeStruct((N_VEH, 1), jnp.int32),
    )(veh_scores.reshape(N_VEH, 1), veh_scores.reshape(1, N_VEH))

    comp_kernel = functools.partial(
        pl.kernel,
        out_type=jax.ShapeDtypeStruct((N_VEH - N_INF, D), jnp.float32),
        mesh=plsc.VectorSubcoreMesh(core_axis_name="c", subcore_axis_name="s"),
        compiler_params=pltpu.CompilerParams(needs_layout_passes=False),
        scratch_types=[
            pltpu.VMEM((N_VEH,), jnp.int32),
            pltpu.VMEM((_CPW,), jnp.int32),
            pltpu.VMEM((_CPW, D), jnp.float32),
            pltpu.SemaphoreType.DMA,
        ],
    )(_sc_comp)
    out_comp = comp_kernel(veh_query_feats, ranks.reshape(N_VEH))

    bestval, bestidx, addv = pl.pallas_call(
        _stage_cost,
        grid=(GRID_A,),
        in_specs=[
            pl.BlockSpec((N_INF, 3), lambda g: (0, 0)),
            pl.BlockSpec((3, BV), lambda g: (0, g)),
            pl.BlockSpec((3, BV), lambda g: (0, g)),
            pl.BlockSpec((1, BV), lambda g: (0, g)),
            pl.BlockSpec((N_INF, D), lambda g: (0, 0)),
            pl.BlockSpec((D, D), lambda g: (0, 0)),
            pl.BlockSpec((1, D), lambda g: (0, 0)),
        ],
        out_specs=[
            pl.BlockSpec((N_INF, 1), lambda g: (0, 0)),
            pl.BlockSpec((N_INF, 1), lambda g: (0, 0)),
            pl.BlockSpec((N_INF, D), lambda g: (0, 0)),
        ],
        out_shape=[
            jax.ShapeDtypeStruct((N_INF, 1), jnp.float32),
            jax.ShapeDtypeStruct((N_INF, 1), jnp.int32),
            jax.ShapeDtypeStruct((N_INF, D), jnp.float32),
        ],
    )(inf_abs, veh_ref_pts.T, veh_pred_dims.T, veh_scores.reshape(1, N_VEH),
      inf_query_feats, W_fusion.T, b_fusion.reshape(1, D))

    fused_kernel = functools.partial(
        pl.kernel,
        out_type=jax.ShapeDtypeStruct((N_INF, D), jnp.float32),
        mesh=plsc.VectorSubcoreMesh(core_axis_name="c", subcore_axis_name="s"),
        compiler_params=pltpu.CompilerParams(needs_layout_passes=False),
        scratch_types=[
            pltpu.VMEM((_FPW,), jnp.int32),
            pltpu.VMEM((_FPW, D), jnp.float32),
            pltpu.VMEM((_FPW, D), jnp.float32),
            pltpu.SemaphoreType.DMA,
            pltpu.SemaphoreType.DMA,
            pltpu.SemaphoreType.DMA,
        ],
    )(_sc_fused)
    out_fused = fused_kernel(veh_query_feats, bestidx.reshape(N_INF), addv)

    return jnp.concatenate([out_fused, out_comp], axis=0)
